# baseline (speedup 1.0000x reference)
import functools

import jax
import jax.numpy as jnp
from jax import lax
from jax.experimental import pallas as pl
from jax.experimental.pallas import tpu as pltpu
from jax.experimental.pallas import tpu_sc as plsc

B, H, W, C = 4, 224, 224, 192
NPIX = B * H * W
NROW = B * H
NW = 32
ROWS_PER_W = NROW // NW
PCHUNK = 32
CHUNKS_PER_ROW = W // PCHUNK
NCHUNK = ROWS_PER_W * CHUNKS_PER_ROW
PIX_PER_W = ROWS_PER_W * W
OSTRIDE = 232


def _lane_bcast(v, lane_scalar):
    idx = jnp.broadcast_to(lane_scalar, (16,))[:, None]
    dnums = lax.GatherDimensionNumbers(
        offset_dims=(), collapsed_slice_dims=(0,), start_index_map=(0,)
    )
    return lax.gather(v, idx, dnums, slice_sizes=(1,),
                      mode=lax.GatherScatterMode.PROMISE_IN_BOUNDS)


def _body(src_hbm, flow_hbm, out_hbm, flow_v, idxa, idxb, rowsa, rowsb,
          orow_v, gsema, gsemb):
    wid = lax.axis_index("s") * 2 + lax.axis_index("c")
    iota = lax.iota(jnp.int32, 16)

    pltpu.sync_copy(flow_hbm.at[pl.ds(wid * (2 * PIX_PER_W), 2 * PIX_PER_W)],
                    flow_v)

    def tap_math(c, g):
        jx = c % CHUNKS_PER_ROW
        img_row = wid * ROWS_PER_W + c // CHUNKS_PER_ROW
        b = img_row // H
        y = img_row % H
        p_local = c * PCHUNK + g * 16 + iota
        fy = plsc.load_gather(flow_v, [2 * p_local])
        fx = plsc.load_gather(flow_v, [2 * p_local + 1])
        xi = jx * PCHUNK + g * 16 + iota
        yfv = jnp.broadcast_to(y.astype(jnp.float32), (16,))
        sy = jnp.clip(yfv + fy, 0.0, float(H - 1))
        sx = jnp.clip(xi.astype(jnp.float32) + fx, 0.0, float(W - 1))
        y0 = sy.astype(jnp.int32)
        x0 = sx.astype(jnp.int32)
        dy = sy - y0.astype(jnp.float32)
        dx = sx - x0.astype(jnp.float32)
        y1 = jnp.minimum(y0 + 1, H - 1)
        x1 = jnp.minimum(x0 + 1, W - 1)
        basev = jnp.broadcast_to(b * (H * W), (16,))
        r00 = basev + y0 * W + x0
        r01 = basev + y0 * W + x1
        r10 = basev + y1 * W + x0
        r11 = basev + y1 * W + x1
        return (r00, r01, r10, r11), dy, dx

    def fire(c, idxr, rows, gsem):
        for g in range(2):
            taps, _, _ = tap_math(c, g)
            for k in range(4):
                idxr[pl.ds(g * 64 + k * 16, 16)] = taps[k]
        pltpu.async_copy(src_hbm.at[idxr], rows, gsem)

    def compute(c, idxr, rows, gsem):
        jx = c % CHUNKS_PER_ROW
        wts = []
        for g in range(2):
            _, dy, dx = tap_math(c, g)
            omy = 1.0 - dy
            omx = 1.0 - dx
            wts.append([omx * omy, dx * omy, omx * dy, dx * dy])
        pltpu.make_async_copy(src_hbm.at[idxr], rows, gsem).wait()

        for g in range(2):
            w = wts[g]
            xbase_g = jx * PCHUNK + g * 16

            @plsc.parallel_loop(0, 16, unroll=8)
            def _px(pp, g=g, w=w, xbase_g=xbase_g):
                wk = [_lane_bcast(w[k], pp) for k in range(4)]
                rb = [g * 64 + k * 16 + pp for k in range(4)]
                xs = jnp.broadcast_to(xbase_g + pp, (16,))
                for v in range(C // 16):
                    acc = wk[0] * rows[rb[0], pl.ds(16 * v, 16)]
                    acc += wk[1] * rows[rb[1], pl.ds(16 * v, 16)]
                    acc += wk[2] * rows[rb[2], pl.ds(16 * v, 16)]
                    acc += wk[3] * rows[rb[3], pl.ds(16 * v, 16)]
                    plsc.store_scatter(orow_v, [16 * v + iota, xs], acc)

        @pl.when(c % CHUNKS_PER_ROW == CHUNKS_PER_ROW - 1)
        def _store():
            img_row = wid * ROWS_PER_W + c // CHUNKS_PER_ROW
            pltpu.sync_copy(orow_v.at[:, pl.ds(0, W)],
                            out_hbm.at[pl.ds(img_row * C, C)])

    def stage(c, idx_cur, rows_cur, gsem_cur, idx_nxt, rows_nxt, gsem_nxt):
        @pl.when(c + 1 < NCHUNK)
        def _prefetch():
            fire(c + 1, idx_nxt, rows_nxt, gsem_nxt)

        compute(c, idx_cur, rows_cur, gsem_cur)

    fire(0, idxa, rowsa, gsema)

    @pl.loop(0, NCHUNK, step=2)
    def _iter(c):
        stage(c, idxa, rowsa, gsema, idxb, rowsb, gsemb)
        stage(c + 1, idxb, rowsb, gsemb, idxa, rowsa, gsema)


_warp = functools.partial(
    pl.kernel,
    out_type=jax.ShapeDtypeStruct((NROW * C, W), jnp.float32),
    mesh=plsc.VectorSubcoreMesh(
        core_axis_name="c", subcore_axis_name="s", num_cores=2, num_subcores=16
    ),
    compiler_params=pltpu.CompilerParams(
        needs_layout_passes=False, use_tc_tiling_on_sc=False
    ),
    scratch_types=[
        pltpu.VMEM((2 * PIX_PER_W,), jnp.float32),
        pltpu.VMEM((128,), jnp.int32),
        pltpu.VMEM((128,), jnp.int32),
        pltpu.VMEM((128, C), jnp.float32),
        pltpu.VMEM((128, C), jnp.float32),
        pltpu.VMEM((C, OSTRIDE), jnp.float32),
        pltpu.SemaphoreType.DMA,
        pltpu.SemaphoreType.DMA,
    ],
)(_body)


@jax.jit
def kernel(src, flow):
    src2 = src.reshape(NPIX, C)
    flow2 = flow.reshape(NPIX * 2)
    out3 = _warp(src2, flow2)
    return out3.reshape(B, H, C, W).transpose(0, 1, 3, 2)


# speedup vs baseline: 1.4349x; 1.4349x over previous
import functools

import jax
import jax.numpy as jnp
from jax import lax
from jax.experimental import pallas as pl
from jax.experimental.pallas import tpu as pltpu
from jax.experimental.pallas import tpu_sc as plsc

B, H, W, C = 4, 224, 224, 192
NB = 2
NPIX = NB * H * W
NROW = NB * H
NW = 32
ROWS_PER_W = NROW // NW
PCHUNK = 32
CHUNKS_PER_ROW = W // PCHUNK
NCHUNK = ROWS_PER_W * CHUNKS_PER_ROW
PIX_PER_W = ROWS_PER_W * W
OSTRIDE = 232


def _lane_bcast(v, lane_scalar):
    idx = jnp.broadcast_to(lane_scalar, (16,))[:, None]
    dnums = lax.GatherDimensionNumbers(
        offset_dims=(), collapsed_slice_dims=(0,), start_index_map=(0,)
    )
    return lax.gather(v, idx, dnums, slice_sizes=(1,),
                      mode=lax.GatherScatterMode.PROMISE_IN_BOUNDS)


def _body(src_hbm, flow_hbm, out_hbm, flow_v, idxa, idxb, rowsa, rowsb,
          orow_v, gsema, gsemb):
    wid = lax.axis_index("s") * 2 + lax.axis_index("c")
    iota = lax.iota(jnp.int32, 16)

    pltpu.sync_copy(flow_hbm.at[pl.ds(wid * (2 * PIX_PER_W), 2 * PIX_PER_W)],
                    flow_v)

    def tap_math(c, g):
        jx = c % CHUNKS_PER_ROW
        img_row = wid * ROWS_PER_W + c // CHUNKS_PER_ROW
        b = img_row // H
        y = img_row % H
        p_local = c * PCHUNK + g * 16 + iota
        fy = plsc.load_gather(flow_v, [2 * p_local])
        fx = plsc.load_gather(flow_v, [2 * p_local + 1])
        xi = jx * PCHUNK + g * 16 + iota
        yfv = jnp.broadcast_to(y.astype(jnp.float32), (16,))
        sy = jnp.clip(yfv + fy, 0.0, float(H - 1))
        sx = jnp.clip(xi.astype(jnp.float32) + fx, 0.0, float(W - 1))
        y0 = sy.astype(jnp.int32)
        x0 = sx.astype(jnp.int32)
        dy = sy - y0.astype(jnp.float32)
        dx = sx - x0.astype(jnp.float32)
        y1 = jnp.minimum(y0 + 1, H - 1)
        x1 = jnp.minimum(x0 + 1, W - 1)
        basev = jnp.broadcast_to(b * (H * W), (16,))
        r00 = basev + y0 * W + x0
        r01 = basev + y0 * W + x1
        r10 = basev + y1 * W + x0
        r11 = basev + y1 * W + x1
        return (r00, r01, r10, r11), dy, dx

    def fire(c, idxr, rows, gsem):
        for g in range(2):
            taps, _, _ = tap_math(c, g)
            for k in range(4):
                idxr[pl.ds(g * 64 + k * 16, 16)] = taps[k]
        pltpu.async_copy(src_hbm.at[idxr], rows, gsem)

    def compute(c, idxr, rows, gsem):
        jx = c % CHUNKS_PER_ROW
        wts = []
        for g in range(2):
            _, dy, dx = tap_math(c, g)
            omy = 1.0 - dy
            omx = 1.0 - dx
            wts.append([omx * omy, dx * omy, omx * dy, dx * dy])
        pltpu.make_async_copy(src_hbm.at[idxr], rows, gsem).wait()

        for g in range(2):
            w = wts[g]
            xbase_g = jx * PCHUNK + g * 16

            @plsc.parallel_loop(0, 16, unroll=4)
            def _px(pp, g=g, w=w, xbase_g=xbase_g):
                wk = [_lane_bcast(w[k], pp) for k in range(4)]
                rb = [g * 64 + k * 16 + pp for k in range(4)]
                xs = jnp.broadcast_to(xbase_g + pp, (16,))
                for v in range(C // 16):
                    acc = wk[0] * rows[rb[0], pl.ds(16 * v, 16)]
                    acc += wk[1] * rows[rb[1], pl.ds(16 * v, 16)]
                    acc += wk[2] * rows[rb[2], pl.ds(16 * v, 16)]
                    acc += wk[3] * rows[rb[3], pl.ds(16 * v, 16)]
                    plsc.store_scatter(orow_v, [16 * v + iota, xs], acc)

        @pl.when(c % CHUNKS_PER_ROW == CHUNKS_PER_ROW - 1)
        def _store():
            img_row = wid * ROWS_PER_W + c // CHUNKS_PER_ROW
            pltpu.sync_copy(orow_v.at[:, pl.ds(0, W)],
                            out_hbm.at[pl.ds(img_row * C, C)])

    def stage(c, idx_cur, rows_cur, gsem_cur, idx_nxt, rows_nxt, gsem_nxt):
        @pl.when(c + 1 < NCHUNK)
        def _prefetch():
            fire(c + 1, idx_nxt, rows_nxt, gsem_nxt)

        compute(c, idx_cur, rows_cur, gsem_cur)

    fire(0, idxa, rowsa, gsema)

    @pl.loop(0, NCHUNK, step=2)
    def _iter(c):
        stage(c, idxa, rowsa, gsema, idxb, rowsb, gsemb)
        stage(c + 1, idxb, rowsb, gsemb, idxa, rowsa, gsema)


_warp = functools.partial(
    pl.kernel,
    out_type=jax.ShapeDtypeStruct((NROW * C, W), jnp.float32),
    mesh=plsc.VectorSubcoreMesh(
        core_axis_name="c", subcore_axis_name="s", num_cores=2, num_subcores=16
    ),
    compiler_params=pltpu.CompilerParams(
        needs_layout_passes=False, use_tc_tiling_on_sc=False
    ),
    scratch_types=[
        pltpu.VMEM((2 * PIX_PER_W,), jnp.float32),
        pltpu.VMEM((128,), jnp.int32),
        pltpu.VMEM((128,), jnp.int32),
        pltpu.VMEM((128, C), jnp.float32),
        pltpu.VMEM((128, C), jnp.float32),
        pltpu.VMEM((C, OSTRIDE), jnp.float32),
        pltpu.SemaphoreType.DMA,
        pltpu.SemaphoreType.DMA,
    ],
)(_body)


@jax.jit
def kernel(src, flow):
    outs = []
    for i in range(B // NB):
        s = src[i * NB:(i + 1) * NB].reshape(NPIX, C)
        f = flow[i * NB:(i + 1) * NB].reshape(NPIX * 2)
        o = _warp(s, f)
        outs.append(o.reshape(NB, H, C, W).transpose(0, 1, 3, 2))
    return jnp.concatenate(outs, axis=0)


# speedup vs baseline: 1.6042x; 1.1180x over previous
import functools

import jax
import jax.numpy as jnp
from jax import lax
from jax.experimental import pallas as pl
from jax.experimental.pallas import tpu as pltpu
from jax.experimental.pallas import tpu_sc as plsc

B, H, W, C = 4, 224, 224, 192
NB = 2
NPIX = NB * H * W
NROW = NB * H
NW = 32
ROWS_PER_W = NROW // NW
PCHUNK = 32
CHUNKS_PER_ROW = W // PCHUNK
NCHUNK = ROWS_PER_W * CHUNKS_PER_ROW
PIX_PER_W = ROWS_PER_W * W
OSTRIDE = 232


def _lane_bcast(v, lane_scalar):
    idx = jnp.broadcast_to(lane_scalar, (16,))[:, None]
    dnums = lax.GatherDimensionNumbers(
        offset_dims=(), collapsed_slice_dims=(0,), start_index_map=(0,)
    )
    return lax.gather(v, idx, dnums, slice_sizes=(1,),
                      mode=lax.GatherScatterMode.PROMISE_IN_BOUNDS)


def _body(src_hbm, flow_hbm, out_hbm, flow_v, idxa, idxb, rowsa, rowsb,
          orow_v, gsema, gsemb):
    wid = lax.axis_index("s") * 2 + lax.axis_index("c")
    iota = lax.iota(jnp.int32, 16)

    pltpu.sync_copy(flow_hbm.at[pl.ds(wid * (2 * PIX_PER_W), 2 * PIX_PER_W)],
                    flow_v)

    def tap_math(c, g):
        jx = c % CHUNKS_PER_ROW
        img_row = wid * ROWS_PER_W + c // CHUNKS_PER_ROW
        b = img_row // H
        y = img_row % H
        foff = (c // CHUNKS_PER_ROW) * (2 * W) + jx * PCHUNK + g * 16
        fy = flow_v[pl.ds(foff, 16)]
        fx = flow_v[pl.ds(foff + W, 16)]
        xi = jx * PCHUNK + g * 16 + iota
        yfv = jnp.broadcast_to(y.astype(jnp.float32), (16,))
        sy = jnp.clip(yfv + fy, 0.0, float(H - 1))
        sx = jnp.clip(xi.astype(jnp.float32) + fx, 0.0, float(W - 1))
        y0 = sy.astype(jnp.int32)
        x0 = sx.astype(jnp.int32)
        dy = sy - y0.astype(jnp.float32)
        dx = sx - x0.astype(jnp.float32)
        y1 = jnp.minimum(y0 + 1, H - 1)
        x1 = jnp.minimum(x0 + 1, W - 1)
        basev = jnp.broadcast_to(b * (H * W), (16,))
        r00 = basev + y0 * W + x0
        r01 = basev + y0 * W + x1
        r10 = basev + y1 * W + x0
        r11 = basev + y1 * W + x1
        return (r00, r01, r10, r11), dy, dx

    def fire(c, idxr, rows, gsem):
        for g in range(2):
            taps, _, _ = tap_math(c, g)
            for k in range(4):
                idxr[pl.ds(g * 64 + k * 16, 16)] = taps[k]
        pltpu.async_copy(src_hbm.at[idxr], rows, gsem)

    def compute(c, idxr, rows, gsem):
        jx = c % CHUNKS_PER_ROW
        wts = []
        for g in range(2):
            _, dy, dx = tap_math(c, g)
            omy = 1.0 - dy
            omx = 1.0 - dx
            wts.append([omx * omy, dx * omy, omx * dy, dx * dy])
        pltpu.make_async_copy(src_hbm.at[idxr], rows, gsem).wait()

        for g in range(2):
            w = wts[g]
            xbase_g = jx * PCHUNK + g * 16

            @plsc.parallel_loop(0, 16, unroll=4)
            def _px(pp, g=g, w=w, xbase_g=xbase_g):
                wk = [_lane_bcast(w[k], pp) for k in range(4)]
                rb = [g * 64 + k * 16 + pp for k in range(4)]
                xs = jnp.broadcast_to(xbase_g + pp, (16,))
                for v in range(C // 16):
                    acc = wk[0] * rows[rb[0], pl.ds(16 * v, 16)]
                    acc += wk[1] * rows[rb[1], pl.ds(16 * v, 16)]
                    acc += wk[2] * rows[rb[2], pl.ds(16 * v, 16)]
                    acc += wk[3] * rows[rb[3], pl.ds(16 * v, 16)]
                    plsc.store_scatter(orow_v, [16 * v + iota, xs], acc)

        @pl.when(c % CHUNKS_PER_ROW == CHUNKS_PER_ROW - 1)
        def _store():
            img_row = wid * ROWS_PER_W + c // CHUNKS_PER_ROW
            pltpu.sync_copy(orow_v.at[:, pl.ds(0, W)],
                            out_hbm.at[pl.ds(img_row * C, C)])

    def stage(c, idx_cur, rows_cur, gsem_cur, idx_nxt, rows_nxt, gsem_nxt):
        @pl.when(c + 1 < NCHUNK)
        def _prefetch():
            fire(c + 1, idx_nxt, rows_nxt, gsem_nxt)

        compute(c, idx_cur, rows_cur, gsem_cur)

    fire(0, idxa, rowsa, gsema)

    @pl.loop(0, NCHUNK, step=2)
    def _iter(c):
        stage(c, idxa, rowsa, gsema, idxb, rowsb, gsemb)
        stage(c + 1, idxb, rowsb, gsemb, idxa, rowsa, gsema)


_warp = functools.partial(
    pl.kernel,
    out_type=jax.ShapeDtypeStruct((NROW * C, W), jnp.float32),
    mesh=plsc.VectorSubcoreMesh(
        core_axis_name="c", subcore_axis_name="s", num_cores=2, num_subcores=16
    ),
    compiler_params=pltpu.CompilerParams(
        needs_layout_passes=False, use_tc_tiling_on_sc=False
    ),
    scratch_types=[
        pltpu.VMEM((2 * PIX_PER_W,), jnp.float32),
        pltpu.VMEM((128,), jnp.int32),
        pltpu.VMEM((128,), jnp.int32),
        pltpu.VMEM((128, C), jnp.float32),
        pltpu.VMEM((128, C), jnp.float32),
        pltpu.VMEM((C, OSTRIDE), jnp.float32),
        pltpu.SemaphoreType.DMA,
        pltpu.SemaphoreType.DMA,
    ],
)(_body)


@jax.jit
def kernel(src, flow):
    outs = []
    for i in range(B // NB):
        s = src[i * NB:(i + 1) * NB].reshape(NPIX, C)
        f = flow[i * NB:(i + 1) * NB].transpose(0, 1, 3, 2).reshape(NPIX * 2)
        o = _warp(s, f)
        outs.append(o.reshape(NB, H, C, W).transpose(0, 1, 3, 2))
    return jnp.concatenate(outs, axis=0)


# speedup vs baseline: 1.6780x; 1.0460x over previous
import functools

import jax
import jax.numpy as jnp
from jax import lax
from jax.experimental import pallas as pl
from jax.experimental.pallas import tpu as pltpu
from jax.experimental.pallas import tpu_sc as plsc

B, H, W, C = 4, 224, 224, 192
NB = 1
NPIX = NB * H * W
NROW = NB * H
NW = 32
ROWS_PER_W = NROW // NW
PCHUNK = 32
CHUNKS_PER_ROW = W // PCHUNK
NCHUNK = ROWS_PER_W * CHUNKS_PER_ROW
PIX_PER_W = ROWS_PER_W * W
OSTRIDE = 232


def _lane_bcast(v, lane_scalar):
    idx = jnp.broadcast_to(lane_scalar, (16,))[:, None]
    dnums = lax.GatherDimensionNumbers(
        offset_dims=(), collapsed_slice_dims=(0,), start_index_map=(0,)
    )
    return lax.gather(v, idx, dnums, slice_sizes=(1,),
                      mode=lax.GatherScatterMode.PROMISE_IN_BOUNDS)


def _body(src_hbm, flow_hbm, out_hbm, flow_v, idxa, idxb, rowsa, rowsb,
          orow_v, gsema, gsemb):
    wid = lax.axis_index("s") * 2 + lax.axis_index("c")
    iota = lax.iota(jnp.int32, 16)

    pltpu.sync_copy(flow_hbm.at[pl.ds(wid * (2 * PIX_PER_W), 2 * PIX_PER_W)],
                    flow_v)

    def tap_math(c, g):
        jx = c % CHUNKS_PER_ROW
        img_row = wid * ROWS_PER_W + c // CHUNKS_PER_ROW
        b = img_row // H
        y = img_row % H
        foff = (c // CHUNKS_PER_ROW) * (2 * W) + jx * PCHUNK + g * 16
        fy = flow_v[pl.ds(foff, 16)]
        fx = flow_v[pl.ds(foff + W, 16)]
        xi = jx * PCHUNK + g * 16 + iota
        yfv = jnp.broadcast_to(y.astype(jnp.float32), (16,))
        sy = jnp.clip(yfv + fy, 0.0, float(H - 1))
        sx = jnp.clip(xi.astype(jnp.float32) + fx, 0.0, float(W - 1))
        y0 = sy.astype(jnp.int32)
        x0 = sx.astype(jnp.int32)
        dy = sy - y0.astype(jnp.float32)
        dx = sx - x0.astype(jnp.float32)
        y1 = jnp.minimum(y0 + 1, H - 1)
        x1 = jnp.minimum(x0 + 1, W - 1)
        basev = jnp.broadcast_to(b * (H * W), (16,))
        r00 = basev + y0 * W + x0
        r01 = basev + y0 * W + x1
        r10 = basev + y1 * W + x0
        r11 = basev + y1 * W + x1
        return (r00, r01, r10, r11), dy, dx

    def fire(c, idxr, rows, gsem):
        for g in range(2):
            taps, _, _ = tap_math(c, g)
            for k in range(4):
                idxr[pl.ds(g * 64 + k * 16, 16)] = taps[k]
        pltpu.async_copy(src_hbm.at[idxr], rows, gsem)

    def compute(c, idxr, rows, gsem):
        jx = c % CHUNKS_PER_ROW
        wts = []
        for g in range(2):
            _, dy, dx = tap_math(c, g)
            omy = 1.0 - dy
            omx = 1.0 - dx
            wts.append([omx * omy, dx * omy, omx * dy, dx * dy])
        pltpu.make_async_copy(src_hbm.at[idxr], rows, gsem).wait()

        for g in range(2):
            w = wts[g]
            xbase_g = jx * PCHUNK + g * 16

            @plsc.parallel_loop(0, 16, unroll=4)
            def _px(pp, g=g, w=w, xbase_g=xbase_g):
                wk = [_lane_bcast(w[k], pp) for k in range(4)]
                rb = [g * 64 + k * 16 + pp for k in range(4)]
                xs = jnp.broadcast_to(xbase_g + pp, (16,))
                for v in range(C // 16):
                    acc = wk[0] * rows[rb[0], pl.ds(16 * v, 16)]
                    acc += wk[1] * rows[rb[1], pl.ds(16 * v, 16)]
                    acc += wk[2] * rows[rb[2], pl.ds(16 * v, 16)]
                    acc += wk[3] * rows[rb[3], pl.ds(16 * v, 16)]
                    plsc.store_scatter(orow_v, [16 * v + iota, xs], acc)

        @pl.when(c % CHUNKS_PER_ROW == CHUNKS_PER_ROW - 1)
        def _store():
            img_row = wid * ROWS_PER_W + c // CHUNKS_PER_ROW
            pltpu.sync_copy(orow_v.at[:, pl.ds(0, W)],
                            out_hbm.at[pl.ds(img_row * C, C)])

    def stage(c, idx_cur, rows_cur, gsem_cur, idx_nxt, rows_nxt, gsem_nxt):
        @pl.when(c + 1 < NCHUNK)
        def _prefetch():
            fire(c + 1, idx_nxt, rows_nxt, gsem_nxt)

        @pl.when(c < NCHUNK)
        def _compute():
            compute(c, idx_cur, rows_cur, gsem_cur)

    fire(0, idxa, rowsa, gsema)

    @pl.loop(0, NCHUNK + (NCHUNK % 2), step=2)
    def _iter(c):
        stage(c, idxa, rowsa, gsema, idxb, rowsb, gsemb)
        stage(c + 1, idxb, rowsb, gsemb, idxa, rowsa, gsema)


_warp = functools.partial(
    pl.kernel,
    out_type=jax.ShapeDtypeStruct((NROW * C, W), jnp.float32),
    mesh=plsc.VectorSubcoreMesh(
        core_axis_name="c", subcore_axis_name="s", num_cores=2, num_subcores=16
    ),
    compiler_params=pltpu.CompilerParams(
        needs_layout_passes=False, use_tc_tiling_on_sc=False
    ),
    scratch_types=[
        pltpu.VMEM((2 * PIX_PER_W,), jnp.float32),
        pltpu.VMEM((128,), jnp.int32),
        pltpu.VMEM((128,), jnp.int32),
        pltpu.VMEM((128, C), jnp.float32),
        pltpu.VMEM((128, C), jnp.float32),
        pltpu.VMEM((C, OSTRIDE), jnp.float32),
        pltpu.SemaphoreType.DMA,
        pltpu.SemaphoreType.DMA,
    ],
)(_body)


@jax.jit
def kernel(src, flow):
    outs = []
    for i in range(B // NB):
        s = src[i * NB:(i + 1) * NB].reshape(NPIX, C)
        f = flow[i * NB:(i + 1) * NB].transpose(0, 1, 3, 2).reshape(NPIX * 2)
        o = _warp(s, f)
        outs.append(o.reshape(NB, H, C, W).transpose(0, 1, 3, 2))
    return jnp.concatenate(outs, axis=0)
